# baseline (speedup 1.0000x reference)
import jax
import jax.numpy as jnp
from jax.experimental import pallas as pl

B, T, DIN, DH, DM, K = 16, 4096, 32, 64, 128, 32
SEG = T // K
PACK = 4
TP = T // PACK
SEGP = SEG // PACK


def _mlp_pool_kernel(x_ref, w1_ref, b1_ref, w2_ref, b2_ref, gate_ref, o_ref):
    x = x_ref[0]
    h = jnp.dot(x, w1_ref[...], preferred_element_type=jnp.float32) + b1_ref[...]
    h = 0.5 * h * (1.0 + jax.lax.erf(h * jnp.float32(0.7071067811865476)))
    s = jnp.sum(h.reshape(K, SEGP, PACK * DH), axis=1) * jnp.float32(1.0 / SEG)
    y = jnp.dot(s, w2_ref[...], preferred_element_type=jnp.float32) + b2_ref[...]
    o_ref[0] = y * jnp.tanh(gate_ref[0, 0])


def kernel(imu_seq, W1, b1, W2, b2, gate):
    xp = imu_seq.reshape(B, TP, PACK * DIN)
    w1b = jax.scipy.linalg.block_diag(*([W1] * PACK))
    b1r = jnp.tile(b1, PACK).reshape(1, PACK * DH)
    w2r = jnp.concatenate([W2] * PACK, axis=0)
    b2r = b2.reshape(1, DM)
    gr = gate.reshape(1, 1)
    out = pl.pallas_call(
        _mlp_pool_kernel,
        grid=(B,),
        in_specs=[
            pl.BlockSpec((1, TP, PACK * DIN), lambda b: (b, 0, 0)),
            pl.BlockSpec((PACK * DIN, PACK * DH), lambda b: (0, 0)),
            pl.BlockSpec((1, PACK * DH), lambda b: (0, 0)),
            pl.BlockSpec((PACK * DH, DM), lambda b: (0, 0)),
            pl.BlockSpec((1, DM), lambda b: (0, 0)),
            pl.BlockSpec((1, 1), lambda b: (0, 0)),
        ],
        out_specs=pl.BlockSpec((1, K, DM), lambda b: (b, 0, 0)),
        out_shape=jax.ShapeDtypeStruct((B, K, DM), jnp.float32),
    )(xp, w1b, b1r, w2r, b2r, gr)
    return out


# speedup vs baseline: 3.0707x; 3.0707x over previous
import jax
import jax.numpy as jnp
from jax.experimental import pallas as pl

B, T, DIN, DH, DM, K = 16, 4096, 32, 64, 128, 32
SEG = T // K


def _mlp_pool_kernel(x_ref, w1t_ref, b1_ref, w2_ref, b2_ref, gate_ref, o_ref):
    x = x_ref[0]
    h = jnp.dot(w1t_ref[...], x, preferred_element_type=jnp.float32) + b1_ref[...]
    h = 0.5 * h * (1.0 + jax.lax.erf(h * jnp.float32(0.7071067811865476)))
    row = jax.lax.broadcasted_iota(jnp.int32, (T, K), 0)
    col = jax.lax.broadcasted_iota(jnp.int32, (T, K), 1)
    p = jnp.where(row // SEG == col, jnp.float32(1.0 / SEG), jnp.float32(0.0))
    s = jnp.dot(h, p, preferred_element_type=jnp.float32)
    y = jnp.dot(s.T, w2_ref[...], preferred_element_type=jnp.float32) + b2_ref[...]
    o_ref[0] = y * jnp.tanh(gate_ref[0, 0])


def kernel(imu_seq, W1, b1, W2, b2, gate):
    xt = jnp.swapaxes(imu_seq, 1, 2)
    w1t = W1.T
    b1r = b1.reshape(DH, 1)
    b2r = b2.reshape(1, DM)
    gr = gate.reshape(1, 1)
    out = pl.pallas_call(
        _mlp_pool_kernel,
        grid=(B,),
        in_specs=[
            pl.BlockSpec((1, DIN, T), lambda b: (b, 0, 0)),
            pl.BlockSpec((DH, DIN), lambda b: (0, 0)),
            pl.BlockSpec((DH, 1), lambda b: (0, 0)),
            pl.BlockSpec((DH, DM), lambda b: (0, 0)),
            pl.BlockSpec((1, DM), lambda b: (0, 0)),
            pl.BlockSpec((1, 1), lambda b: (0, 0)),
        ],
        out_specs=pl.BlockSpec((1, K, DM), lambda b: (b, 0, 0)),
        out_shape=jax.ShapeDtypeStruct((B, K, DM), jnp.float32),
    )(xt, w1t, b1r, W2, b2r, gr)
    return out


# speedup vs baseline: 3.3298x; 1.0844x over previous
import jax
import jax.numpy as jnp
from jax.experimental import pallas as pl

B, T, DIN, DH, DM, K = 16, 4096, 32, 64, 128, 32
SEG = T // K
NB = 4
GRID = B // NB


def _mlp_pool_kernel(x_ref, w1_ref, b1_ref, w2_ref, b2_ref, g_ref, p_ref, o_ref):
    x = x_ref[...].reshape(NB * DIN, T)
    h = jnp.dot(w1_ref[...], x, preferred_element_type=jnp.float32) + b1_ref[...]
    h = 0.5 * h * (1.0 + jax.lax.erf(h * jnp.float32(0.7071067811865476)))
    s = jnp.dot(h, p_ref[...], preferred_element_type=jnp.float32)
    y = jnp.dot(s.T, w2_ref[...], preferred_element_type=jnp.float32)
    y = (y + b2_ref[...]) * jnp.tanh(g_ref[0, 0])
    for bi in range(NB):
        o_ref[bi] = y[:, bi * DM : (bi + 1) * DM]


def kernel(imu_seq, W1, b1, W2, b2, gate):
    xt = jnp.swapaxes(imu_seq, 1, 2)
    w1bd = jax.scipy.linalg.block_diag(*([W1.T] * NB))
    b1r = jnp.tile(b1, NB).reshape(NB * DH, 1)
    w2bd = jax.scipy.linalg.block_diag(*([W2] * NB))
    b2r = jnp.tile(b2, NB).reshape(1, NB * DM)
    gr = gate.reshape(1, 1)
    t_idx = jnp.arange(T, dtype=jnp.int32)
    pool = (
        jax.nn.one_hot(t_idx // SEG, K, dtype=jnp.float32) * (1.0 / SEG)
    )
    out = pl.pallas_call(
        _mlp_pool_kernel,
        grid=(GRID,),
        in_specs=[
            pl.BlockSpec((NB, DIN, T), lambda g: (g, 0, 0)),
            pl.BlockSpec((NB * DH, NB * DIN), lambda g: (0, 0)),
            pl.BlockSpec((NB * DH, 1), lambda g: (0, 0)),
            pl.BlockSpec((NB * DH, NB * DM), lambda g: (0, 0)),
            pl.BlockSpec((1, NB * DM), lambda g: (0, 0)),
            pl.BlockSpec((1, 1), lambda g: (0, 0)),
            pl.BlockSpec((T, K), lambda g: (0, 0)),
        ],
        out_specs=pl.BlockSpec((NB, K, DM), lambda g: (g, 0, 0)),
        out_shape=jax.ShapeDtypeStruct((B, K, DM), jnp.float32),
    )(xt, w1bd, b1r, w2bd, b2r, gr, pool)
    return out
